# baseline (speedup 1.0000x reference)
import functools

import jax
import jax.numpy as jnp
from jax import lax
from jax.experimental import pallas as pl
from jax.experimental.pallas import tpu as pltpu
from jax.experimental.pallas import tpu_sc as plsc

LATENT = 16
NEMB = 8192
F32 = jnp.float32


def _encoder_kernel(xq, w1, b1, w2, b2, w3, b3, out, he, ho):

  def row(g, c):

    @pl.when(g == 0)
    def _():
      he[0] = jnp.zeros((32, 57), F32)
      ho[0] = jnp.zeros((32, 57), F32)

    parts = []
    for k in range(4):
      r = 4 * g + k
      even_cols = []
      odd_cols = []
      for ky in range(3):
        xr = 2 * r + ky
        for kx in range(3):
          even_cols.append(xq[0, kx, pl.ds(xr, 1), 0:56])
        odd_cols.append(xq[0, 2, pl.ds(xr, 1), 0:56])
        odd_cols.append(xq[0, 3, pl.ds(xr, 1), 0:56])
        odd_cols.append(xq[0, 0, pl.ds(xr, 1), 1:57])
      parts.append(jnp.concatenate(even_cols, axis=0))
      parts.append(jnp.concatenate(odd_cols, axis=0))
    big = jnp.concatenate(parts, axis=1)
    h1 = jnp.maximum(jnp.dot(w1[...], big, preferred_element_type=F32)
                     + b1[...], 0.0)
    zero1 = jnp.zeros((32, 1), F32)
    for k in range(4):
      rp = 4 * g + k + 1
      ho[rp, :, 0:56] = h1[:, (2 * k) * 56:(2 * k + 1) * 56]
      ho[rp, :, 56:57] = zero1
      he[rp, :, 1:57] = h1[:, (2 * k + 1) * 56:(2 * k + 2) * 56]
      he[rp, :, 0:1] = zero1

    halves = []
    for i2 in (2 * g, 2 * g + 1):
      cols = []
      for ky in range(3):
        rp = 2 * i2 + ky
        cols.append(he[rp, :, 0:56])
        cols.append(ho[rp, :, 0:56])
        cols.append(he[rp, :, 1:57])
      halves.append(jnp.concatenate(cols, axis=0))
    patch = jnp.concatenate(halves, axis=1)
    h = jnp.maximum(jnp.dot(w2[...], patch, preferred_element_type=F32)
                    + b2[...], 0.0)
    z = jnp.dot(w3[...], h, preferred_element_type=F32) + b3[...]
    out[0, 2 * g] = z[:, 0:56]
    out[0, 2 * g + 1] = z[:, 56:112]
    return c

  lax.fori_loop(0, 28, row, 0)


def _encoder(xq, w1, b1, w2, b2, w3, b3):
  return pl.pallas_call(
      _encoder_kernel,
      grid=(4,),
      in_specs=[pl.BlockSpec((1, 4, 226, 57), lambda n: (n, 0, 0, 0)),
                pl.BlockSpec((32, 9), lambda n: (0, 0)),
                pl.BlockSpec((32, 1), lambda n: (0, 0)),
                pl.BlockSpec((64, 288), lambda n: (0, 0)),
                pl.BlockSpec((64, 1), lambda n: (0, 0)),
                pl.BlockSpec((16, 64), lambda n: (0, 0)),
                pl.BlockSpec((16, 1), lambda n: (0, 0))],
      out_specs=pl.BlockSpec((1, 56, 16, 56), lambda n: (n, 0, 0, 0)),
      out_shape=jax.ShapeDtypeStruct((4, 56, 16, 56), F32),
      scratch_shapes=[pltpu.VMEM((114, 32, 57), F32),
                      pltpu.VMEM((114, 32, 57), F32)],
  )(xq, w1, b1, w2, b2, w3, b3)


_VQ_ROWS = 256
_VQ_CHUNK = 512


def _vq_kernel(faug, embta, oidx, oloss):
  fb = faug[...]
  best_v = jnp.full((_VQ_ROWS, 1), jnp.inf, F32)
  best_i = jnp.zeros((_VQ_ROWS, 1), jnp.int32)
  iota = lax.broadcasted_iota(jnp.int32, (_VQ_ROWS, _VQ_CHUNK), 1)
  for c in range(NEMB // _VQ_CHUNK):
    ec = embta[:, c * _VQ_CHUNK:(c + 1) * _VQ_CHUNK]
    d = jnp.dot(fb, ec, preferred_element_type=F32)
    m = jnp.min(d, axis=1, keepdims=True)
    cand = jnp.where(d <= m, iota, jnp.int32(2**30))
    ci = jnp.min(cand, axis=1, keepdims=True) + c * _VQ_CHUNK
    upd = m < best_v
    best_i = jnp.where(upd, ci, best_i)
    best_v = jnp.minimum(best_v, m)
  oidx[...] = best_i
  partial = jnp.sum(best_v) + jnp.sum(fb * fb) - F32(_VQ_ROWS)

  @pl.when(pl.program_id(0) == 0)
  def _():
    oloss[0, 0] = partial

  @pl.when(pl.program_id(0) != 0)
  def _():
    oloss[0, 0] = oloss[0, 0] + partial


def _vq_argmin(faug, embta):
  n = faug.shape[0]
  return pl.pallas_call(
      _vq_kernel,
      grid=(n // _VQ_ROWS,),
      in_specs=[pl.BlockSpec((_VQ_ROWS, 17), lambda i: (i, 0)),
                pl.BlockSpec((17, NEMB), lambda i: (0, 0))],
      out_specs=[pl.BlockSpec((_VQ_ROWS, 1), lambda i: (i, 0)),
                 pl.BlockSpec(memory_space=pltpu.SMEM)],
      out_shape=[jax.ShapeDtypeStruct((n, 1), jnp.int32),
                 jax.ShapeDtypeStruct((1, 1), F32)],
  )(faug, embta)


def _gather_sc(emb128, idx):
  n = idx.shape[0]
  info = plsc.get_sparse_core_info()
  nw = info.num_cores * info.num_subcores
  b_per_w = n // nw
  mesh = plsc.VectorSubcoreMesh(core_axis_name="c", subcore_axis_name="s")

  @functools.partial(
      pl.kernel,
      mesh=mesh,
      out_type=jax.ShapeDtypeStruct((n, 128), F32),
      scratch_types=[
          pltpu.VMEM((b_per_w,), jnp.int32),
          pltpu.VMEM((b_per_w, 128), F32),
          pltpu.SemaphoreType.DMA,
      ],
  )
  def gather(table_hbm, idx_hbm, out_hbm, idx_v, rows_v, sem):
    wid = lax.axis_index("s") * info.num_cores + lax.axis_index("c")
    base = wid * b_per_w
    pltpu.sync_copy(idx_hbm.at[pl.ds(base, b_per_w)], idx_v)
    pltpu.async_copy(table_hbm.at[idx_v], rows_v, sem).wait()
    pltpu.sync_copy(rows_v, out_hbm.at[pl.ds(base, b_per_w)])

  return gather(emb128, idx)[:, :LATENT]


def _make_convt_kernel(rows, width, rb):

  def kern(xp, w, b, out):

    def rowgrp(g, c):
      cols = []
      for k in range(rb):
        i = rb * g + k
        cols.append(jnp.concatenate(
            [xp[0, i, :, 0:width], xp[0, i, :, 1:width + 1],
             xp[0, i + 1, :, 0:width], xp[0, i + 1, :, 1:width + 1]],
            axis=0))
      patch = jnp.concatenate(cols, axis=1)
      r = jnp.maximum(
          jnp.dot(w[...], patch, preferred_element_type=F32) + b[...], 0.0)
      for k in range(rb):
        out[0, rb * g + k] = r[:, k * width:(k + 1) * width]
      return c

    lax.fori_loop(0, rows // rb, rowgrp, 0)

  return kern


def _convt(xp, w_all, b_all, rows, width, cout4, rb):
  return pl.pallas_call(
      _make_convt_kernel(rows, width, rb),
      grid=(4,),
      in_specs=[
          pl.BlockSpec((1,) + xp.shape[1:], lambda n: (n, 0, 0, 0)),
          pl.BlockSpec(w_all.shape, lambda n: (0, 0)),
          pl.BlockSpec((cout4, 1), lambda n: (0, 0)),
      ],
      out_specs=pl.BlockSpec((1, rows, cout4, width), lambda n: (n, 0, 0, 0)),
      out_shape=jax.ShapeDtypeStruct((4, rows, cout4, width), F32),
  )(xp, w_all, b_all)


def _pack_convt_w(w):
  cin, cout = w.shape[0], w.shape[1]
  m = lambda a, bb: w[:, :, 2 - a, 2 - bb]
  z = jnp.zeros((cin, cout), F32)
  r0 = jnp.concatenate([m(1, 1), m(1, 0), m(0, 1), m(0, 0)], axis=1)
  r1 = jnp.concatenate([z, m(1, 2), z, m(0, 2)], axis=1)
  r2 = jnp.concatenate([z, z, m(2, 1), m(2, 0)], axis=1)
  r3 = jnp.concatenate([z, z, z, m(2, 2)], axis=1)
  return jnp.concatenate([r0, r1, r2, r3], axis=0)


def _interleave(t, rows, cout, width):
  t = t.reshape(4, rows, 2, 2, cout, width)
  t = jnp.transpose(t, (0, 1, 2, 4, 5, 3))
  return t.reshape(4, 2 * rows, cout, 2 * width)


def _conv3f_kernel(xp, w, b, out):

  def rowgrp(g, c):
    grp = []
    for p in range(3):
      i = 3 * g + p
      cols = []
      for ky in range(3):
        for kx in range(3):
          cols.append(xp[0, i + ky, :, kx:kx + 222])
      grp.append(jnp.concatenate(cols, axis=0))
    patch = jnp.concatenate(grp, axis=1)
    v = jnp.dot(w[...], patch, preferred_element_type=F32) + b[...]
    out[0, pl.ds(g, 1), :] = 1.0 / (1.0 + jnp.exp(-v))
    return c

  lax.fori_loop(0, 74, rowgrp, 0)


def _conv3f(xp, w, b):
  return pl.pallas_call(
      _conv3f_kernel,
      grid=(4,),
      in_specs=[pl.BlockSpec((1, 224, 32, 224), lambda n: (n, 0, 0, 0)),
                pl.BlockSpec((1, 288), lambda n: (0, 0)),
                pl.BlockSpec((1, 1), lambda n: (0, 0))],
      out_specs=pl.BlockSpec((1, 74, 666), lambda n: (n, 0, 0)),
      out_shape=jax.ShapeDtypeStruct((4, 74, 666), F32),
  )(xp, w, b)


@jax.jit
def kernel(x, enc_w1, enc_b1, enc_w2, enc_b2, enc_w3, enc_b3,
           dec_w1, dec_b1, dec_w2, dec_b2, dec_w3, dec_b3, embeddings):
  xp = jnp.pad(x[:, 0, :, :], ((0, 0), (1, 1), (1, 3)))
  xq = jnp.transpose(xp.reshape(4, 226, 57, 4), (0, 3, 1, 2))
  z = _encoder(xq, enc_w1.reshape(32, 9), enc_b1.reshape(32, 1),
               jnp.transpose(enc_w2.reshape(64, 32, 9),
                             (0, 2, 1)).reshape(64, 288),
               enc_b2.reshape(64, 1), enc_w3.reshape(16, 64),
               enc_b3.reshape(16, 1))

  flat = jnp.transpose(z, (0, 1, 3, 2)).reshape(-1, LATENT)
  faug = jnp.pad(flat, ((0, 0), (0, 1)), constant_values=1.0)
  embta = jnp.concatenate(
      [-2.0 * jnp.transpose(embeddings, (1, 0)),
       jnp.sum(embeddings * embeddings, axis=1)[None, :]], axis=0)
  idx2, s = _vq_argmin(faug, embta)
  idx = idx2.reshape(-1)
  emb128 = jnp.pad(embeddings, ((0, 0), (0, 128 - LATENT)))
  q = _gather_sc(emb128, idx)
  vq_loss = 1.25 * s[0, 0] / flat.size

  qt = jnp.pad(jnp.transpose(q.reshape(4, 56, 56, LATENT), (0, 1, 3, 2)),
               ((0, 0), (0, 1), (0, 0), (0, 1)))
  t1 = _convt(qt, jnp.transpose(_pack_convt_w(dec_w1), (1, 0)),
              jnp.tile(dec_b1, 4).reshape(256, 1), 56, 56, 256, 4)
  d1 = _interleave(t1, 56, 64, 56)
  d1p = jnp.pad(d1[:, :111, :, :111], ((0, 0), (0, 1), (0, 0), (0, 1)))

  t2 = _convt(d1p, jnp.transpose(_pack_convt_w(dec_w2), (1, 0)),
              jnp.tile(dec_b2, 4).reshape(128, 1), 111, 111, 128, 3)
  d2s = jnp.pad(_interleave(t2, 111, 32, 111),
                ((0, 0), (1, 1), (0, 0), (1, 1)))

  w3f = jnp.transpose(dec_w3[0], (1, 2, 0)).reshape(1, 288)
  out = _conv3f(d2s, w3f, dec_b3.reshape(1, 1))
  x_recon = out.reshape(4, 1, 222, 222)
  return (x_recon, vq_loss)


# speedup vs baseline: 1.1318x; 1.1318x over previous
import functools

import jax
import jax.numpy as jnp
from jax import lax
from jax.experimental import pallas as pl
from jax.experimental.pallas import tpu as pltpu
from jax.experimental.pallas import tpu_sc as plsc

LATENT = 16
NEMB = 8192
F32 = jnp.float32


def _encoder_kernel(xq, w1, b1, w2, b2, w3, b3, out, he, ho):

  def row(g, c):

    @pl.when(g == 0)
    def _():
      he[0] = jnp.zeros((32, 57), F32)
      ho[0] = jnp.zeros((32, 57), F32)

    parts = []
    for k in range(4):
      r = 4 * g + k
      even_cols = []
      odd_cols = []
      for ky in range(3):
        xr = 2 * r + ky
        for kx in range(3):
          even_cols.append(xq[0, kx, pl.ds(xr, 1), 0:56])
        odd_cols.append(xq[0, 2, pl.ds(xr, 1), 0:56])
        odd_cols.append(xq[0, 3, pl.ds(xr, 1), 0:56])
        odd_cols.append(xq[0, 0, pl.ds(xr, 1), 1:57])
      parts.append(jnp.concatenate(even_cols, axis=0))
      parts.append(jnp.concatenate(odd_cols, axis=0))
    big = jnp.concatenate(parts, axis=1)
    h1 = jnp.maximum(jnp.dot(w1[...], big, preferred_element_type=F32)
                     + b1[...], 0.0)
    zero1 = jnp.zeros((32, 1), F32)
    for k in range(4):
      rp = 4 * g + k + 1
      ho[rp, :, 0:56] = h1[:, (2 * k) * 56:(2 * k + 1) * 56]
      ho[rp, :, 56:57] = zero1
      he[rp, :, 1:57] = h1[:, (2 * k + 1) * 56:(2 * k + 2) * 56]
      he[rp, :, 0:1] = zero1

    halves = []
    for i2 in (2 * g, 2 * g + 1):
      cols = []
      for ky in range(3):
        rp = 2 * i2 + ky
        cols.append(he[rp, :, 0:56])
        cols.append(ho[rp, :, 0:56])
        cols.append(he[rp, :, 1:57])
      halves.append(jnp.concatenate(cols, axis=0))
    patch = jnp.concatenate(halves, axis=1)
    h = jnp.maximum(jnp.dot(w2[...], patch, preferred_element_type=F32)
                    + b2[...], 0.0)
    z = jnp.dot(w3[...], h, preferred_element_type=F32) + b3[...]
    out[0, 2 * g] = z[:, 0:56]
    out[0, 2 * g + 1] = z[:, 56:112]
    return c

  lax.fori_loop(0, 28, row, 0)


def _encoder(xq, w1, b1, w2, b2, w3, b3):
  return pl.pallas_call(
      _encoder_kernel,
      grid=(4,),
      in_specs=[pl.BlockSpec((1, 4, 226, 57), lambda n: (n, 0, 0, 0)),
                pl.BlockSpec((32, 9), lambda n: (0, 0)),
                pl.BlockSpec((32, 1), lambda n: (0, 0)),
                pl.BlockSpec((64, 288), lambda n: (0, 0)),
                pl.BlockSpec((64, 1), lambda n: (0, 0)),
                pl.BlockSpec((16, 64), lambda n: (0, 0)),
                pl.BlockSpec((16, 1), lambda n: (0, 0))],
      out_specs=pl.BlockSpec((1, 56, 16, 56), lambda n: (n, 0, 0, 0)),
      out_shape=jax.ShapeDtypeStruct((4, 56, 16, 56), F32),
      scratch_shapes=[pltpu.VMEM((114, 32, 57), F32),
                      pltpu.VMEM((114, 32, 57), F32)],
  )(xq, w1, b1, w2, b2, w3, b3)


_VQ_ROWS = 256
_VQ_CHUNK = 512


def _vq_kernel(faug, embta, oidx, oloss):
  fb = faug[...]
  best_v = jnp.full((_VQ_ROWS, 1), jnp.inf, F32)
  best_i = jnp.zeros((_VQ_ROWS, 1), jnp.int32)
  iota = lax.broadcasted_iota(jnp.int32, (_VQ_ROWS, _VQ_CHUNK), 1)
  for c in range(NEMB // _VQ_CHUNK):
    ec = embta[:, c * _VQ_CHUNK:(c + 1) * _VQ_CHUNK]
    d = jnp.dot(fb, ec, preferred_element_type=F32)
    m = jnp.min(d, axis=1, keepdims=True)
    cand = jnp.where(d <= m, iota, jnp.int32(2**30))
    ci = jnp.min(cand, axis=1, keepdims=True) + c * _VQ_CHUNK
    upd = m < best_v
    best_i = jnp.where(upd, ci, best_i)
    best_v = jnp.minimum(best_v, m)
  oidx[...] = best_i
  partial = jnp.sum(best_v) + jnp.sum(fb * fb) - F32(_VQ_ROWS)

  @pl.when(pl.program_id(0) == 0)
  def _():
    oloss[0, 0] = partial

  @pl.when(pl.program_id(0) != 0)
  def _():
    oloss[0, 0] = oloss[0, 0] + partial


def _vq_argmin(faug, embta):
  n = faug.shape[0]
  return pl.pallas_call(
      _vq_kernel,
      grid=(n // _VQ_ROWS,),
      in_specs=[pl.BlockSpec((_VQ_ROWS, 17), lambda i: (i, 0)),
                pl.BlockSpec((17, NEMB), lambda i: (0, 0))],
      out_specs=[pl.BlockSpec((_VQ_ROWS, 1), lambda i: (i, 0)),
                 pl.BlockSpec(memory_space=pltpu.SMEM)],
      out_shape=[jax.ShapeDtypeStruct((n, 1), jnp.int32),
                 jax.ShapeDtypeStruct((1, 1), F32)],
  )(faug, embta)


def _gather_sc(emb, idx):
  n = idx.shape[0]
  info = plsc.get_sparse_core_info()
  nw = info.num_cores * info.num_subcores
  b_per_w = n // nw
  mesh = plsc.VectorSubcoreMesh(core_axis_name="c", subcore_axis_name="s")

  @functools.partial(
      pl.kernel,
      mesh=mesh,
      out_type=jax.ShapeDtypeStruct((n, LATENT), F32),
      compiler_params=pltpu.CompilerParams(use_tc_tiling_on_sc=False),
      scratch_types=[
          pltpu.VMEM((b_per_w,), jnp.int32),
          pltpu.VMEM((b_per_w, LATENT), F32),
          pltpu.SemaphoreType.DMA,
      ],
  )
  def gather(table_hbm, idx_hbm, out_hbm, idx_v, rows_v, sem):
    wid = lax.axis_index("s") * info.num_cores + lax.axis_index("c")
    base = wid * b_per_w
    pltpu.sync_copy(idx_hbm.at[pl.ds(base, b_per_w)], idx_v)
    pltpu.async_copy(table_hbm.at[idx_v], rows_v, sem).wait()
    pltpu.sync_copy(rows_v, out_hbm.at[pl.ds(base, b_per_w)])

  return gather(emb, idx)


def _make_convt_kernel(rows, width, rb):

  def kern(xp, w, b, out):

    def rowgrp(g, c):
      cols = []
      for k in range(rb):
        i = rb * g + k
        cols.append(jnp.concatenate(
            [xp[0, i, :, 0:width], xp[0, i, :, 1:width + 1],
             xp[0, i + 1, :, 0:width], xp[0, i + 1, :, 1:width + 1]],
            axis=0))
      patch = jnp.concatenate(cols, axis=1)
      r = jnp.maximum(
          jnp.dot(w[...], patch, preferred_element_type=F32) + b[...], 0.0)
      for k in range(rb):
        out[0, rb * g + k] = r[:, k * width:(k + 1) * width]
      return c

    lax.fori_loop(0, rows // rb, rowgrp, 0)

  return kern


def _convt(xp, w_all, b_all, rows, width, cout4, rb):
  return pl.pallas_call(
      _make_convt_kernel(rows, width, rb),
      grid=(4,),
      in_specs=[
          pl.BlockSpec((1,) + xp.shape[1:], lambda n: (n, 0, 0, 0)),
          pl.BlockSpec(w_all.shape, lambda n: (0, 0)),
          pl.BlockSpec((cout4, 1), lambda n: (0, 0)),
      ],
      out_specs=pl.BlockSpec((1, rows, cout4, width), lambda n: (n, 0, 0, 0)),
      out_shape=jax.ShapeDtypeStruct((4, rows, cout4, width), F32),
  )(xp, w_all, b_all)


def _pack_convt_w(w):
  cin, cout = w.shape[0], w.shape[1]
  m = lambda a, bb: w[:, :, 2 - a, 2 - bb]
  z = jnp.zeros((cin, cout), F32)
  r0 = jnp.concatenate([m(1, 1), m(1, 0), m(0, 1), m(0, 0)], axis=1)
  r1 = jnp.concatenate([z, m(1, 2), z, m(0, 2)], axis=1)
  r2 = jnp.concatenate([z, z, m(2, 1), m(2, 0)], axis=1)
  r3 = jnp.concatenate([z, z, z, m(2, 2)], axis=1)
  return jnp.concatenate([r0, r1, r2, r3], axis=0)


def _interleave(t, rows, cout, width):
  t = t.reshape(4, rows, 2, 2, cout, width)
  t = jnp.transpose(t, (0, 1, 2, 4, 5, 3))
  return t.reshape(4, 2 * rows, cout, 2 * width)


def _conv3f_kernel(xp, w, b, out):

  def rowgrp(g, c):
    grp = []
    for p in range(3):
      i = 3 * g + p
      cols = []
      for ky in range(3):
        for kx in range(3):
          cols.append(xp[0, i + ky, :, kx:kx + 222])
      grp.append(jnp.concatenate(cols, axis=0))
    patch = jnp.concatenate(grp, axis=1)
    v = jnp.dot(w[...], patch, preferred_element_type=F32) + b[...]
    out[0, pl.ds(g, 1), :] = 1.0 / (1.0 + jnp.exp(-v))
    return c

  lax.fori_loop(0, 74, rowgrp, 0)


def _conv3f(xp, w, b):
  return pl.pallas_call(
      _conv3f_kernel,
      grid=(4,),
      in_specs=[pl.BlockSpec((1, 224, 32, 224), lambda n: (n, 0, 0, 0)),
                pl.BlockSpec((1, 288), lambda n: (0, 0)),
                pl.BlockSpec((1, 1), lambda n: (0, 0))],
      out_specs=pl.BlockSpec((1, 74, 666), lambda n: (n, 0, 0)),
      out_shape=jax.ShapeDtypeStruct((4, 74, 666), F32),
  )(xp, w, b)


@jax.jit
def kernel(x, enc_w1, enc_b1, enc_w2, enc_b2, enc_w3, enc_b3,
           dec_w1, dec_b1, dec_w2, dec_b2, dec_w3, dec_b3, embeddings):
  xp = jnp.pad(x[:, 0, :, :], ((0, 0), (1, 1), (1, 3)))
  xq = jnp.transpose(xp.reshape(4, 226, 57, 4), (0, 3, 1, 2))
  z = _encoder(xq, enc_w1.reshape(32, 9), enc_b1.reshape(32, 1),
               jnp.transpose(enc_w2.reshape(64, 32, 9),
                             (0, 2, 1)).reshape(64, 288),
               enc_b2.reshape(64, 1), enc_w3.reshape(16, 64),
               enc_b3.reshape(16, 1))

  flat = jnp.transpose(z, (0, 1, 3, 2)).reshape(-1, LATENT)
  faug = jnp.pad(flat, ((0, 0), (0, 1)), constant_values=1.0)
  embta = jnp.concatenate(
      [-2.0 * jnp.transpose(embeddings, (1, 0)),
       jnp.sum(embeddings * embeddings, axis=1)[None, :]], axis=0)
  idx2, s = _vq_argmin(faug, embta)
  idx = idx2.reshape(-1)
  q = _gather_sc(embeddings, idx)
  vq_loss = 1.25 * s[0, 0] / flat.size

  qt = jnp.pad(jnp.transpose(q.reshape(4, 56, 56, LATENT), (0, 1, 3, 2)),
               ((0, 0), (0, 1), (0, 0), (0, 1)))
  t1 = _convt(qt, jnp.transpose(_pack_convt_w(dec_w1), (1, 0)),
              jnp.tile(dec_b1, 4).reshape(256, 1), 56, 56, 256, 4)
  d1 = _interleave(t1, 56, 64, 56)
  d1p = jnp.pad(d1[:, :111, :, :111], ((0, 0), (0, 1), (0, 0), (0, 1)))

  t2 = _convt(d1p, jnp.transpose(_pack_convt_w(dec_w2), (1, 0)),
              jnp.tile(dec_b2, 4).reshape(128, 1), 111, 111, 128, 3)
  d2s = jnp.pad(_interleave(t2, 111, 32, 111),
                ((0, 0), (1, 1), (0, 0), (1, 1)))

  w3f = jnp.transpose(dec_w3[0], (1, 2, 0)).reshape(1, 288)
  out = _conv3f(d2s, w3f, dec_b3.reshape(1, 1))
  x_recon = out.reshape(4, 1, 222, 222)
  return (x_recon, vq_loss)


# speedup vs baseline: 1.2028x; 1.0627x over previous
import functools

import jax
import jax.numpy as jnp
from jax import lax
from jax.experimental import pallas as pl
from jax.experimental.pallas import tpu as pltpu
from jax.experimental.pallas import tpu_sc as plsc

LATENT = 16
NEMB = 8192
F32 = jnp.float32


def _encoder_kernel(xq, w1, b1, w2, b2, w3, b3, out, he, ho):

  def row(g, c):

    @pl.when(g == 0)
    def _():
      he[0] = jnp.zeros((32, 57), F32)
      ho[0] = jnp.zeros((32, 57), F32)

    parts = []
    for k in range(4):
      r = 4 * g + k
      even_cols = []
      odd_cols = []
      for ky in range(3):
        xr = 2 * r + ky
        for kx in range(3):
          even_cols.append(xq[0, kx, pl.ds(xr, 1), 0:56])
        odd_cols.append(xq[0, 2, pl.ds(xr, 1), 0:56])
        odd_cols.append(xq[0, 3, pl.ds(xr, 1), 0:56])
        odd_cols.append(xq[0, 0, pl.ds(xr, 1), 1:57])
      parts.append(jnp.concatenate(even_cols, axis=0))
      parts.append(jnp.concatenate(odd_cols, axis=0))
    big = jnp.concatenate(parts, axis=1)
    h1 = jnp.maximum(jnp.dot(w1[...], big, preferred_element_type=F32)
                     + b1[...], 0.0)
    zero1 = jnp.zeros((32, 1), F32)
    for k in range(4):
      rp = 4 * g + k + 1
      ho[rp, :, 0:56] = h1[:, (2 * k) * 56:(2 * k + 1) * 56]
      ho[rp, :, 56:57] = zero1
      he[rp, :, 1:57] = h1[:, (2 * k + 1) * 56:(2 * k + 2) * 56]
      he[rp, :, 0:1] = zero1

    halves = []
    for i2 in (2 * g, 2 * g + 1):
      cols = []
      for ky in range(3):
        rp = 2 * i2 + ky
        cols.append(he[rp, :, 0:56])
        cols.append(ho[rp, :, 0:56])
        cols.append(he[rp, :, 1:57])
      halves.append(jnp.concatenate(cols, axis=0))
    patch = jnp.concatenate(halves, axis=1)
    h = jnp.maximum(jnp.dot(w2[...], patch, preferred_element_type=F32)
                    + b2[...], 0.0)
    z = jnp.dot(w3[...], h, preferred_element_type=F32) + b3[...]
    out[0, 2 * g] = z[:, 0:56]
    out[0, 2 * g + 1] = z[:, 56:112]
    return c

  lax.fori_loop(0, 28, row, 0)


def _encoder(xq, w1, b1, w2, b2, w3, b3):
  return pl.pallas_call(
      _encoder_kernel,
      grid=(4,),
      in_specs=[pl.BlockSpec((1, 4, 226, 57), lambda n: (n, 0, 0, 0)),
                pl.BlockSpec((32, 9), lambda n: (0, 0)),
                pl.BlockSpec((32, 1), lambda n: (0, 0)),
                pl.BlockSpec((64, 288), lambda n: (0, 0)),
                pl.BlockSpec((64, 1), lambda n: (0, 0)),
                pl.BlockSpec((16, 64), lambda n: (0, 0)),
                pl.BlockSpec((16, 1), lambda n: (0, 0))],
      out_specs=pl.BlockSpec((1, 56, 16, 56), lambda n: (n, 0, 0, 0)),
      out_shape=jax.ShapeDtypeStruct((4, 56, 16, 56), F32),
      scratch_shapes=[pltpu.VMEM((114, 32, 57), F32),
                      pltpu.VMEM((114, 32, 57), F32)],
  )(xq, w1, b1, w2, b2, w3, b3)


_VQ_ROWS = 256
_VQ_CHUNK = 512


def _vq_kernel(faug, embta, oidx, oloss):
  fb = faug[...]
  best_v = jnp.full((_VQ_ROWS, 1), jnp.inf, F32)
  best_i = jnp.zeros((_VQ_ROWS, 1), jnp.int32)
  iota = lax.broadcasted_iota(jnp.int32, (_VQ_ROWS, _VQ_CHUNK), 1)
  for c in range(NEMB // _VQ_CHUNK):
    ec = embta[:, c * _VQ_CHUNK:(c + 1) * _VQ_CHUNK]
    d = jnp.dot(fb, ec, preferred_element_type=F32)
    m = jnp.min(d, axis=1, keepdims=True)
    cand = jnp.where(d <= m, iota, jnp.int32(2**30))
    ci = jnp.min(cand, axis=1, keepdims=True) + c * _VQ_CHUNK
    upd = m < best_v
    best_i = jnp.where(upd, ci, best_i)
    best_v = jnp.minimum(best_v, m)
  oidx[...] = best_i
  partial = jnp.sum(best_v) + jnp.sum(fb * fb) - F32(_VQ_ROWS)

  @pl.when(pl.program_id(0) == 0)
  def _():
    oloss[0, 0] = partial

  @pl.when(pl.program_id(0) != 0)
  def _():
    oloss[0, 0] = oloss[0, 0] + partial


def _vq_argmin(faug, embta):
  n = faug.shape[0]
  return pl.pallas_call(
      _vq_kernel,
      grid=(n // _VQ_ROWS,),
      in_specs=[pl.BlockSpec((_VQ_ROWS, 17), lambda i: (i, 0)),
                pl.BlockSpec((17, NEMB), lambda i: (0, 0))],
      out_specs=[pl.BlockSpec((_VQ_ROWS, 1), lambda i: (i, 0)),
                 pl.BlockSpec(memory_space=pltpu.SMEM)],
      out_shape=[jax.ShapeDtypeStruct((n, 1), jnp.int32),
                 jax.ShapeDtypeStruct((1, 1), F32)],
  )(faug, embta)


def _gather_sc(emb, idx):
  n = idx.shape[0]
  info = plsc.get_sparse_core_info()
  nw = info.num_cores * info.num_subcores
  b_per_w = n // nw
  mesh = plsc.VectorSubcoreMesh(core_axis_name="c", subcore_axis_name="s")

  @functools.partial(
      pl.kernel,
      mesh=mesh,
      out_type=jax.ShapeDtypeStruct((n, LATENT), F32),
      compiler_params=pltpu.CompilerParams(use_tc_tiling_on_sc=False),
      scratch_types=[
          pltpu.VMEM((b_per_w,), jnp.int32),
          pltpu.VMEM((b_per_w, LATENT), F32),
          pltpu.SemaphoreType.DMA,
      ],
  )
  def gather(table_hbm, idx_hbm, out_hbm, idx_v, rows_v, sem):
    wid = lax.axis_index("s") * info.num_cores + lax.axis_index("c")
    base = wid * b_per_w
    pltpu.sync_copy(idx_hbm.at[pl.ds(base, b_per_w)], idx_v)
    pltpu.async_copy(table_hbm.at[idx_v], rows_v, sem).wait()
    pltpu.sync_copy(rows_v, out_hbm.at[pl.ds(base, b_per_w)])

  return gather(emb, idx)


def _make_convt_kernel(rows, width, rb):

  def kern(xp, w, b, out):

    def rowgrp(g, c):
      cols = []
      for k in range(rb):
        i = rb * g + k
        cols.append(jnp.concatenate(
            [xp[0, i, :, 0:width], xp[0, i, :, 1:width + 1],
             xp[0, i + 1, :, 0:width], xp[0, i + 1, :, 1:width + 1]],
            axis=0))
      patch = jnp.concatenate(cols, axis=1)
      r = jnp.maximum(
          jnp.dot(w[...], patch, preferred_element_type=F32) + b[...], 0.0)
      for k in range(rb):
        out[0, rb * g + k] = r[:, k * width:(k + 1) * width]
      return c

    lax.fori_loop(0, rows // rb, rowgrp, 0)

  return kern


def _convt(xp, w_all, b_all, rows, width, cout4, rb):
  return pl.pallas_call(
      _make_convt_kernel(rows, width, rb),
      grid=(4,),
      in_specs=[
          pl.BlockSpec((1,) + xp.shape[1:], lambda n: (n, 0, 0, 0)),
          pl.BlockSpec(w_all.shape, lambda n: (0, 0)),
          pl.BlockSpec((cout4, 1), lambda n: (0, 0)),
      ],
      out_specs=pl.BlockSpec((1, rows, cout4, width), lambda n: (n, 0, 0, 0)),
      out_shape=jax.ShapeDtypeStruct((4, rows, cout4, width), F32),
  )(xp, w_all, b_all)


def _pack_convt_w(w):
  cin, cout = w.shape[0], w.shape[1]
  m = lambda a, bb: w[:, :, 2 - a, 2 - bb]
  z = jnp.zeros((cin, cout), F32)
  r0 = jnp.concatenate([m(1, 1), m(1, 0), m(0, 1), m(0, 0)], axis=1)
  r1 = jnp.concatenate([z, m(1, 2), z, m(0, 2)], axis=1)
  r2 = jnp.concatenate([z, z, m(2, 1), m(2, 0)], axis=1)
  r3 = jnp.concatenate([z, z, z, m(2, 2)], axis=1)
  return jnp.concatenate([r0, r1, r2, r3], axis=0)


def _interleave(t, rows, cout, width):
  t = t.reshape(4, rows, 2, 2, cout, width)
  t = jnp.transpose(t, (0, 1, 2, 4, 5, 3))
  return t.reshape(4, 2 * rows, cout, 2 * width)


def _conv3f_kernel(xp, w, b, out):

  def rowgrp(g, c):
    grp = []
    for p in range(6):
      i = 6 * g + p
      cols = []
      for ky in range(3):
        for kx in range(3):
          cols.append(xp[0, i + ky, :, kx:kx + 222])
      grp.append(jnp.concatenate(cols, axis=0))
    patch = jnp.concatenate(grp, axis=1)
    v = jnp.dot(w[...], patch, preferred_element_type=F32) + b[...]
    out[0, pl.ds(g, 1), :] = 1.0 / (1.0 + jnp.exp(-v))
    return c

  lax.fori_loop(0, 37, rowgrp, 0)


def _conv3f(xp, w, b):
  return pl.pallas_call(
      _conv3f_kernel,
      grid=(4,),
      in_specs=[pl.BlockSpec((1, 224, 32, 224), lambda n: (n, 0, 0, 0)),
                pl.BlockSpec((1, 288), lambda n: (0, 0)),
                pl.BlockSpec((1, 1), lambda n: (0, 0))],
      out_specs=pl.BlockSpec((1, 37, 1332), lambda n: (n, 0, 0)),
      out_shape=jax.ShapeDtypeStruct((4, 37, 1332), F32),
  )(xp, w, b)


@jax.jit
def kernel(x, enc_w1, enc_b1, enc_w2, enc_b2, enc_w3, enc_b3,
           dec_w1, dec_b1, dec_w2, dec_b2, dec_w3, dec_b3, embeddings):
  xp = jnp.pad(x[:, 0, :, :], ((0, 0), (1, 1), (1, 3)))
  xq = jnp.transpose(xp.reshape(4, 226, 57, 4), (0, 3, 1, 2))
  z = _encoder(xq, enc_w1.reshape(32, 9), enc_b1.reshape(32, 1),
               jnp.transpose(enc_w2.reshape(64, 32, 9),
                             (0, 2, 1)).reshape(64, 288),
               enc_b2.reshape(64, 1), enc_w3.reshape(16, 64),
               enc_b3.reshape(16, 1))

  flat = jnp.transpose(z, (0, 1, 3, 2)).reshape(-1, LATENT)
  faug = jnp.pad(flat, ((0, 0), (0, 1)), constant_values=1.0)
  embta = jnp.concatenate(
      [-2.0 * jnp.transpose(embeddings, (1, 0)),
       jnp.sum(embeddings * embeddings, axis=1)[None, :]], axis=0)
  idx2, s = _vq_argmin(faug, embta)
  idx = idx2.reshape(-1)
  q = _gather_sc(embeddings, idx)
  vq_loss = 1.25 * s[0, 0] / flat.size

  qt = jnp.pad(jnp.transpose(q.reshape(4, 56, 56, LATENT), (0, 1, 3, 2)),
               ((0, 0), (0, 1), (0, 0), (0, 1)))
  t1 = _convt(qt, jnp.transpose(_pack_convt_w(dec_w1), (1, 0)),
              jnp.tile(dec_b1, 4).reshape(256, 1), 56, 56, 256, 8)
  d1 = _interleave(t1, 56, 64, 56)
  d1p = jnp.pad(d1[:, :111, :, :111], ((0, 0), (0, 1), (0, 0), (0, 1)))

  t2 = _convt(d1p, jnp.transpose(_pack_convt_w(dec_w2), (1, 0)),
              jnp.tile(dec_b2, 4).reshape(128, 1), 111, 111, 128, 3)
  d2s = jnp.pad(_interleave(t2, 111, 32, 111),
                ((0, 0), (1, 1), (0, 0), (1, 1)))

  w3f = jnp.transpose(dec_w3[0], (1, 2, 0)).reshape(1, 288)
  out = _conv3f(d2s, w3f, dec_b3.reshape(1, 1))
  x_recon = out.reshape(4, 1, 222, 222)
  return (x_recon, vq_loss)


# speedup vs baseline: 1.3145x; 1.0929x over previous
import functools

import jax
import jax.numpy as jnp
from jax import lax
from jax.experimental import pallas as pl
from jax.experimental.pallas import tpu as pltpu
from jax.experimental.pallas import tpu_sc as plsc

LATENT = 16
NEMB = 8192
F32 = jnp.float32


def _encoder_kernel(xq, w1, b1, w2, b2, w3, b3, out, he, ho):

  def row(g, c):

    @pl.when(g == 0)
    def _():
      he[0] = jnp.zeros((32, 57), F32)
      ho[0] = jnp.zeros((32, 57), F32)

    parts = []
    for k in range(4):
      r = 4 * g + k
      even_cols = []
      odd_cols = []
      for ky in range(3):
        xr = 2 * r + ky
        for kx in range(3):
          even_cols.append(xq[0, kx, pl.ds(xr, 1), 0:56])
        odd_cols.append(xq[0, 2, pl.ds(xr, 1), 0:56])
        odd_cols.append(xq[0, 3, pl.ds(xr, 1), 0:56])
        odd_cols.append(xq[0, 0, pl.ds(xr, 1), 1:57])
      parts.append(jnp.concatenate(even_cols, axis=0))
      parts.append(jnp.concatenate(odd_cols, axis=0))
    big = jnp.concatenate(parts, axis=1)
    h1 = jnp.maximum(jnp.dot(w1[...], big, preferred_element_type=F32)
                     + b1[...], 0.0)
    zero1 = jnp.zeros((32, 1), F32)
    for k in range(4):
      rp = 4 * g + k + 1
      ho[rp, :, 0:56] = h1[:, (2 * k) * 56:(2 * k + 1) * 56]
      ho[rp, :, 56:57] = zero1
      he[rp, :, 1:57] = h1[:, (2 * k + 1) * 56:(2 * k + 2) * 56]
      he[rp, :, 0:1] = zero1

    halves = []
    for i2 in (2 * g, 2 * g + 1):
      cols = []
      for ky in range(3):
        rp = 2 * i2 + ky
        cols.append(he[rp, :, 0:56])
        cols.append(ho[rp, :, 0:56])
        cols.append(he[rp, :, 1:57])
      halves.append(jnp.concatenate(cols, axis=0))
    patch = jnp.concatenate(halves, axis=1)
    h = jnp.maximum(jnp.dot(w2[...], patch, preferred_element_type=F32)
                    + b2[...], 0.0)
    z = jnp.dot(w3[...], h, preferred_element_type=F32) + b3[...]
    out[0, 2 * g] = z[:, 0:56]
    out[0, 2 * g + 1] = z[:, 56:112]
    return c

  lax.fori_loop(0, 28, row, 0)


def _encoder(xq, w1, b1, w2, b2, w3, b3):
  return pl.pallas_call(
      _encoder_kernel,
      grid=(4,),
      in_specs=[pl.BlockSpec((1, 4, 226, 57), lambda n: (n, 0, 0, 0)),
                pl.BlockSpec((32, 9), lambda n: (0, 0)),
                pl.BlockSpec((32, 1), lambda n: (0, 0)),
                pl.BlockSpec((64, 288), lambda n: (0, 0)),
                pl.BlockSpec((64, 1), lambda n: (0, 0)),
                pl.BlockSpec((16, 64), lambda n: (0, 0)),
                pl.BlockSpec((16, 1), lambda n: (0, 0))],
      out_specs=pl.BlockSpec((1, 56, 16, 56), lambda n: (n, 0, 0, 0)),
      out_shape=jax.ShapeDtypeStruct((4, 56, 16, 56), F32),
      scratch_shapes=[pltpu.VMEM((114, 32, 57), F32),
                      pltpu.VMEM((114, 32, 57), F32)],
  )(xq, w1, b1, w2, b2, w3, b3)


_VQ_ROWS = 256
_VQ_CHUNK = 1024


def _vq_kernel(faug, embta, oidx, oloss):
  fb = faug[...]
  best_v = jnp.full((_VQ_ROWS, 1), jnp.inf, F32)
  best_i = jnp.zeros((_VQ_ROWS, 1), jnp.int32)
  iota = lax.broadcasted_iota(jnp.int32, (_VQ_ROWS, _VQ_CHUNK), 1)
  for c in range(NEMB // _VQ_CHUNK):
    ec = embta[:, c * _VQ_CHUNK:(c + 1) * _VQ_CHUNK]
    d = jnp.dot(fb, ec, preferred_element_type=F32)
    m = jnp.min(d, axis=1, keepdims=True)
    cand = jnp.where(d <= m, iota, jnp.int32(2**30))
    ci = jnp.min(cand, axis=1, keepdims=True) + c * _VQ_CHUNK
    upd = m < best_v
    best_i = jnp.where(upd, ci, best_i)
    best_v = jnp.minimum(best_v, m)
  oidx[...] = best_i
  partial = jnp.sum(best_v) + jnp.sum(fb * fb) - F32(_VQ_ROWS)

  @pl.when(pl.program_id(0) == 0)
  def _():
    oloss[0, 0] = partial

  @pl.when(pl.program_id(0) != 0)
  def _():
    oloss[0, 0] = oloss[0, 0] + partial


def _vq_argmin(faug, embta):
  n = faug.shape[0]
  return pl.pallas_call(
      _vq_kernel,
      grid=(n // _VQ_ROWS,),
      in_specs=[pl.BlockSpec((_VQ_ROWS, 17), lambda i: (i, 0)),
                pl.BlockSpec((17, NEMB), lambda i: (0, 0))],
      out_specs=[pl.BlockSpec((_VQ_ROWS, 1), lambda i: (i, 0)),
                 pl.BlockSpec(memory_space=pltpu.SMEM)],
      out_shape=[jax.ShapeDtypeStruct((n, 1), jnp.int32),
                 jax.ShapeDtypeStruct((1, 1), F32)],
  )(faug, embta)


def _gather_sc(emb, idx):
  n = idx.shape[0]
  info = plsc.get_sparse_core_info()
  nw = info.num_cores * info.num_subcores
  b_per_w = n // nw
  mesh = plsc.VectorSubcoreMesh(core_axis_name="c", subcore_axis_name="s")

  @functools.partial(
      pl.kernel,
      mesh=mesh,
      out_type=jax.ShapeDtypeStruct((n, LATENT), F32),
      compiler_params=pltpu.CompilerParams(use_tc_tiling_on_sc=False),
      scratch_types=[
          pltpu.VMEM((b_per_w,), jnp.int32),
          pltpu.VMEM((b_per_w, LATENT), F32),
          pltpu.SemaphoreType.DMA,
      ],
  )
  def gather(table_hbm, idx_hbm, out_hbm, idx_v, rows_v, sem):
    wid = lax.axis_index("s") * info.num_cores + lax.axis_index("c")
    base = wid * b_per_w
    pltpu.sync_copy(idx_hbm.at[pl.ds(base, b_per_w)], idx_v)
    pltpu.async_copy(table_hbm.at[idx_v], rows_v, sem).wait()
    pltpu.sync_copy(rows_v, out_hbm.at[pl.ds(base, b_per_w)])

  return gather(emb, idx)


def _make_convt_kernel(rows, width, rb):

  def kern(xp, w, b, out):

    def rowgrp(g, c):
      cols = []
      for k in range(rb):
        i = rb * g + k
        cols.append(jnp.concatenate(
            [xp[0, i, :, 0:width], xp[0, i, :, 1:width + 1],
             xp[0, i + 1, :, 0:width], xp[0, i + 1, :, 1:width + 1]],
            axis=0))
      patch = jnp.concatenate(cols, axis=1)
      r = jnp.maximum(
          jnp.dot(w[...], patch, preferred_element_type=F32) + b[...], 0.0)
      for k in range(rb):
        out[0, rb * g + k] = r[:, k * width:(k + 1) * width]
      return c

    lax.fori_loop(0, rows // rb, rowgrp, 0)

  return kern


def _convt(xp, w_all, b_all, rows, width, cout4, rb):
  return pl.pallas_call(
      _make_convt_kernel(rows, width, rb),
      grid=(4,),
      in_specs=[
          pl.BlockSpec((1,) + xp.shape[1:], lambda n: (n, 0, 0, 0)),
          pl.BlockSpec(w_all.shape, lambda n: (0, 0)),
          pl.BlockSpec((cout4, 1), lambda n: (0, 0)),
      ],
      out_specs=pl.BlockSpec((1, rows, cout4, width), lambda n: (n, 0, 0, 0)),
      out_shape=jax.ShapeDtypeStruct((4, rows, cout4, width), F32),
  )(xp, w_all, b_all)


def _pack_convt_w(w):
  cin, cout = w.shape[0], w.shape[1]
  m = lambda a, bb: w[:, :, 2 - a, 2 - bb]
  z = jnp.zeros((cin, cout), F32)
  r0 = jnp.concatenate([m(1, 1), m(1, 0), m(0, 1), m(0, 0)], axis=1)
  r1 = jnp.concatenate([z, m(1, 2), z, m(0, 2)], axis=1)
  r2 = jnp.concatenate([z, z, m(2, 1), m(2, 0)], axis=1)
  r3 = jnp.concatenate([z, z, z, m(2, 2)], axis=1)
  return jnp.concatenate([r0, r1, r2, r3], axis=0)


def _interleave(t, rows, cout, width):
  t = t.reshape(4, rows, 2, 2, cout, width)
  t = jnp.transpose(t, (0, 1, 2, 4, 5, 3))
  return t.reshape(4, 2 * rows, cout, 2 * width)


def _conv3f_kernel(xp, w, b, out):

  def rowgrp(g, c):
    grp = []
    for p in range(6):
      i = 6 * g + p
      cols = []
      for ky in range(3):
        for kx in range(3):
          cols.append(xp[0, i + ky, :, kx:kx + 222])
      grp.append(jnp.concatenate(cols, axis=0))
    patch = jnp.concatenate(grp, axis=1)
    v = jnp.dot(w[...], patch, preferred_element_type=F32) + b[...]
    out[0, pl.ds(g, 1), :] = 1.0 / (1.0 + jnp.exp(-v))
    return c

  lax.fori_loop(0, 37, rowgrp, 0)


def _conv3f(xp, w, b):
  return pl.pallas_call(
      _conv3f_kernel,
      grid=(4,),
      in_specs=[pl.BlockSpec((1, 224, 32, 224), lambda n: (n, 0, 0, 0)),
                pl.BlockSpec((1, 288), lambda n: (0, 0)),
                pl.BlockSpec((1, 1), lambda n: (0, 0))],
      out_specs=pl.BlockSpec((1, 37, 1332), lambda n: (n, 0, 0)),
      out_shape=jax.ShapeDtypeStruct((4, 37, 1332), F32),
  )(xp, w, b)


@jax.jit
def kernel(x, enc_w1, enc_b1, enc_w2, enc_b2, enc_w3, enc_b3,
           dec_w1, dec_b1, dec_w2, dec_b2, dec_w3, dec_b3, embeddings):
  xp = jnp.pad(x[:, 0, :, :], ((0, 0), (1, 1), (1, 3)))
  xq = jnp.transpose(xp.reshape(4, 226, 57, 4), (0, 3, 1, 2))
  z = _encoder(xq, enc_w1.reshape(32, 9), enc_b1.reshape(32, 1),
               jnp.transpose(enc_w2.reshape(64, 32, 9),
                             (0, 2, 1)).reshape(64, 288),
               enc_b2.reshape(64, 1), enc_w3.reshape(16, 64),
               enc_b3.reshape(16, 1))

  flat = jnp.transpose(z, (0, 1, 3, 2)).reshape(-1, LATENT)
  faug = jnp.pad(flat, ((0, 0), (0, 1)), constant_values=1.0)
  embta = jnp.concatenate(
      [-2.0 * jnp.transpose(embeddings, (1, 0)),
       jnp.sum(embeddings * embeddings, axis=1)[None, :]], axis=0)
  idx2, s = _vq_argmin(faug, embta)
  idx = idx2.reshape(-1)
  q = _gather_sc(embeddings, idx)
  vq_loss = 1.25 * s[0, 0] / flat.size

  qt = jnp.pad(jnp.transpose(q.reshape(4, 56, 56, LATENT), (0, 1, 3, 2)),
               ((0, 0), (0, 1), (0, 0), (0, 1)))
  t1 = _convt(qt, jnp.transpose(_pack_convt_w(dec_w1), (1, 0)),
              jnp.tile(dec_b1, 4).reshape(256, 1), 56, 56, 256, 8)
  d1 = _interleave(t1, 56, 64, 56)
  d1p = jnp.pad(d1[:, :111, :, :111], ((0, 0), (0, 1), (0, 0), (0, 1)))

  t2 = _convt(d1p, jnp.transpose(_pack_convt_w(dec_w2), (1, 0)),
              jnp.tile(dec_b2, 4).reshape(128, 1), 111, 111, 128, 3)
  d2s = jnp.pad(_interleave(t2, 111, 32, 111),
                ((0, 0), (1, 1), (0, 0), (1, 1)))

  w3f = jnp.transpose(dec_w3[0], (1, 2, 0)).reshape(1, 288)
  out = _conv3f(d2s, w3f, dec_b3.reshape(1, 1))
  x_recon = out.reshape(4, 1, 222, 222)
  return (x_recon, vq_loss)


# speedup vs baseline: 1.3150x; 1.0004x over previous
import functools

import jax
import jax.numpy as jnp
from jax import lax
from jax.experimental import pallas as pl
from jax.experimental.pallas import tpu as pltpu
from jax.experimental.pallas import tpu_sc as plsc

LATENT = 16
NEMB = 8192
F32 = jnp.float32


def _encoder_kernel(xq, w1, b1, w2, b2, w3, b3, out, he, ho):

  def row(g, c):

    @pl.when(g == 0)
    def _():
      he[0] = jnp.zeros((32, 57), F32)
      ho[0] = jnp.zeros((32, 57), F32)

    parts = []
    for k in range(4):
      r = 4 * g + k
      even_cols = []
      odd_cols = []
      for ky in range(3):
        xr = 2 * r + ky
        for kx in range(3):
          even_cols.append(xq[0, kx, pl.ds(xr, 1), 0:56])
        odd_cols.append(xq[0, 2, pl.ds(xr, 1), 0:56])
        odd_cols.append(xq[0, 3, pl.ds(xr, 1), 0:56])
        odd_cols.append(xq[0, 0, pl.ds(xr, 1), 1:57])
      parts.append(jnp.concatenate(even_cols, axis=0))
      parts.append(jnp.concatenate(odd_cols, axis=0))
    big = jnp.concatenate(parts, axis=1)
    h1 = jnp.maximum(jnp.dot(w1[...], big, preferred_element_type=F32)
                     + b1[...], 0.0)
    zero1 = jnp.zeros((32, 1), F32)
    for k in range(4):
      rp = 4 * g + k + 1
      ho[rp, :, 0:56] = h1[:, (2 * k) * 56:(2 * k + 1) * 56]
      ho[rp, :, 56:57] = zero1
      he[rp, :, 1:57] = h1[:, (2 * k + 1) * 56:(2 * k + 2) * 56]
      he[rp, :, 0:1] = zero1

    halves = []
    for i2 in (2 * g, 2 * g + 1):
      cols = []
      for ky in range(3):
        rp = 2 * i2 + ky
        cols.append(he[rp, :, 0:56])
        cols.append(ho[rp, :, 0:56])
        cols.append(he[rp, :, 1:57])
      halves.append(jnp.concatenate(cols, axis=0))
    patch = jnp.concatenate(halves, axis=1)
    h = jnp.maximum(jnp.dot(w2[...], patch, preferred_element_type=F32)
                    + b2[...], 0.0)
    z = jnp.dot(w3[...], h, preferred_element_type=F32) + b3[...]
    out[0, 2 * g] = z[:, 0:56]
    out[0, 2 * g + 1] = z[:, 56:112]
    return c

  lax.fori_loop(0, 28, row, 0)


def _encoder(xq, w1, b1, w2, b2, w3, b3):
  return pl.pallas_call(
      _encoder_kernel,
      grid=(4,),
      in_specs=[pl.BlockSpec((1, 4, 226, 57), lambda n: (n, 0, 0, 0)),
                pl.BlockSpec((32, 9), lambda n: (0, 0)),
                pl.BlockSpec((32, 1), lambda n: (0, 0)),
                pl.BlockSpec((64, 288), lambda n: (0, 0)),
                pl.BlockSpec((64, 1), lambda n: (0, 0)),
                pl.BlockSpec((16, 64), lambda n: (0, 0)),
                pl.BlockSpec((16, 1), lambda n: (0, 0))],
      out_specs=pl.BlockSpec((1, 56, 16, 56), lambda n: (n, 0, 0, 0)),
      out_shape=jax.ShapeDtypeStruct((4, 56, 16, 56), F32),
      scratch_shapes=[pltpu.VMEM((114, 32, 57), F32),
                      pltpu.VMEM((114, 32, 57), F32)],
  )(xq, w1, b1, w2, b2, w3, b3)


_VQ_ROWS = 256
_VQ_CHUNK = 2048


def _vq_kernel(faug, embta, oidx, oloss):
  fb = faug[...]
  best_v = jnp.full((_VQ_ROWS, 1), jnp.inf, F32)
  best_i = jnp.zeros((_VQ_ROWS, 1), jnp.int32)
  iota = lax.broadcasted_iota(jnp.int32, (_VQ_ROWS, _VQ_CHUNK), 1)
  for c in range(NEMB // _VQ_CHUNK):
    ec = embta[:, c * _VQ_CHUNK:(c + 1) * _VQ_CHUNK]
    d = jnp.dot(fb, ec, preferred_element_type=F32)
    m = jnp.min(d, axis=1, keepdims=True)
    cand = jnp.where(d <= m, iota, jnp.int32(2**30))
    ci = jnp.min(cand, axis=1, keepdims=True) + c * _VQ_CHUNK
    upd = m < best_v
    best_i = jnp.where(upd, ci, best_i)
    best_v = jnp.minimum(best_v, m)
  oidx[...] = best_i
  partial = jnp.sum(best_v) + jnp.sum(fb * fb) - F32(_VQ_ROWS)

  @pl.when(pl.program_id(0) == 0)
  def _():
    oloss[0, 0] = partial

  @pl.when(pl.program_id(0) != 0)
  def _():
    oloss[0, 0] = oloss[0, 0] + partial


def _vq_argmin(faug, embta):
  n = faug.shape[0]
  return pl.pallas_call(
      _vq_kernel,
      grid=(n // _VQ_ROWS,),
      in_specs=[pl.BlockSpec((_VQ_ROWS, 17), lambda i: (i, 0)),
                pl.BlockSpec((17, NEMB), lambda i: (0, 0))],
      out_specs=[pl.BlockSpec((_VQ_ROWS, 1), lambda i: (i, 0)),
                 pl.BlockSpec(memory_space=pltpu.SMEM)],
      out_shape=[jax.ShapeDtypeStruct((n, 1), jnp.int32),
                 jax.ShapeDtypeStruct((1, 1), F32)],
  )(faug, embta)


def _gather_sc(emb, idx):
  n = idx.shape[0]
  info = plsc.get_sparse_core_info()
  nw = info.num_cores * info.num_subcores
  b_per_w = n // nw
  mesh = plsc.VectorSubcoreMesh(core_axis_name="c", subcore_axis_name="s")

  @functools.partial(
      pl.kernel,
      mesh=mesh,
      out_type=jax.ShapeDtypeStruct((n, LATENT), F32),
      compiler_params=pltpu.CompilerParams(use_tc_tiling_on_sc=False),
      scratch_types=[
          pltpu.VMEM((b_per_w,), jnp.int32),
          pltpu.VMEM((b_per_w, LATENT), F32),
          pltpu.SemaphoreType.DMA,
      ],
  )
  def gather(table_hbm, idx_hbm, out_hbm, idx_v, rows_v, sem):
    wid = lax.axis_index("s") * info.num_cores + lax.axis_index("c")
    base = wid * b_per_w
    pltpu.sync_copy(idx_hbm.at[pl.ds(base, b_per_w)], idx_v)
    pltpu.async_copy(table_hbm.at[idx_v], rows_v, sem).wait()
    pltpu.sync_copy(rows_v, out_hbm.at[pl.ds(base, b_per_w)])

  return gather(emb, idx)


def _make_convt_kernel(rows, width, rb):

  def kern(xp, w, b, out):

    def rowgrp(g, c):
      cols = []
      for k in range(rb):
        i = rb * g + k
        cols.append(jnp.concatenate(
            [xp[0, i, :, 0:width], xp[0, i, :, 1:width + 1],
             xp[0, i + 1, :, 0:width], xp[0, i + 1, :, 1:width + 1]],
            axis=0))
      patch = jnp.concatenate(cols, axis=1)
      r = jnp.maximum(
          jnp.dot(w[...], patch, preferred_element_type=F32) + b[...], 0.0)
      for k in range(rb):
        out[0, rb * g + k] = r[:, k * width:(k + 1) * width]
      return c

    lax.fori_loop(0, rows // rb, rowgrp, 0)

  return kern


def _convt(xp, w_all, b_all, rows, width, cout4, rb):
  return pl.pallas_call(
      _make_convt_kernel(rows, width, rb),
      grid=(4,),
      in_specs=[
          pl.BlockSpec((1,) + xp.shape[1:], lambda n: (n, 0, 0, 0)),
          pl.BlockSpec(w_all.shape, lambda n: (0, 0)),
          pl.BlockSpec((cout4, 1), lambda n: (0, 0)),
      ],
      out_specs=pl.BlockSpec((1, rows, cout4, width), lambda n: (n, 0, 0, 0)),
      out_shape=jax.ShapeDtypeStruct((4, rows, cout4, width), F32),
  )(xp, w_all, b_all)


def _pack_convt_w(w):
  cin, cout = w.shape[0], w.shape[1]
  m = lambda a, bb: w[:, :, 2 - a, 2 - bb]
  z = jnp.zeros((cin, cout), F32)
  r0 = jnp.concatenate([m(1, 1), m(1, 0), m(0, 1), m(0, 0)], axis=1)
  r1 = jnp.concatenate([z, m(1, 2), z, m(0, 2)], axis=1)
  r2 = jnp.concatenate([z, z, m(2, 1), m(2, 0)], axis=1)
  r3 = jnp.concatenate([z, z, z, m(2, 2)], axis=1)
  return jnp.concatenate([r0, r1, r2, r3], axis=0)


def _interleave(t, rows, cout, width):
  t = t.reshape(4, rows, 2, 2, cout, width)
  t = jnp.transpose(t, (0, 1, 2, 4, 5, 3))
  return t.reshape(4, 2 * rows, cout, 2 * width)


def _conv3f_kernel(xp, w, b, out):

  def rowgrp(g, c):
    grp = []
    for p in range(6):
      i = 6 * g + p
      cols = []
      for ky in range(3):
        for kx in range(3):
          cols.append(xp[0, i + ky, :, kx:kx + 222])
      grp.append(jnp.concatenate(cols, axis=0))
    patch = jnp.concatenate(grp, axis=1)
    v = jnp.dot(w[...], patch, preferred_element_type=F32) + b[...]
    out[0, pl.ds(g, 1), :] = 1.0 / (1.0 + jnp.exp(-v))
    return c

  lax.fori_loop(0, 37, rowgrp, 0)


def _conv3f(xp, w, b):
  return pl.pallas_call(
      _conv3f_kernel,
      grid=(4,),
      in_specs=[pl.BlockSpec((1, 224, 32, 224), lambda n: (n, 0, 0, 0)),
                pl.BlockSpec((1, 288), lambda n: (0, 0)),
                pl.BlockSpec((1, 1), lambda n: (0, 0))],
      out_specs=pl.BlockSpec((1, 37, 1332), lambda n: (n, 0, 0)),
      out_shape=jax.ShapeDtypeStruct((4, 37, 1332), F32),
  )(xp, w, b)


@jax.jit
def kernel(x, enc_w1, enc_b1, enc_w2, enc_b2, enc_w3, enc_b3,
           dec_w1, dec_b1, dec_w2, dec_b2, dec_w3, dec_b3, embeddings):
  xp = jnp.pad(x[:, 0, :, :], ((0, 0), (1, 1), (1, 3)))
  xq = jnp.transpose(xp.reshape(4, 226, 57, 4), (0, 3, 1, 2))
  z = _encoder(xq, enc_w1.reshape(32, 9), enc_b1.reshape(32, 1),
               jnp.transpose(enc_w2.reshape(64, 32, 9),
                             (0, 2, 1)).reshape(64, 288),
               enc_b2.reshape(64, 1), enc_w3.reshape(16, 64),
               enc_b3.reshape(16, 1))

  flat = jnp.transpose(z, (0, 1, 3, 2)).reshape(-1, LATENT)
  faug = jnp.pad(flat, ((0, 0), (0, 1)), constant_values=1.0)
  embta = jnp.concatenate(
      [-2.0 * jnp.transpose(embeddings, (1, 0)),
       jnp.sum(embeddings * embeddings, axis=1)[None, :]], axis=0)
  idx2, s = _vq_argmin(faug, embta)
  idx = idx2.reshape(-1)
  q = _gather_sc(embeddings, idx)
  vq_loss = 1.25 * s[0, 0] / flat.size

  qt = jnp.pad(jnp.transpose(q.reshape(4, 56, 56, LATENT), (0, 1, 3, 2)),
               ((0, 0), (0, 1), (0, 0), (0, 1)))
  t1 = _convt(qt, jnp.transpose(_pack_convt_w(dec_w1), (1, 0)),
              jnp.tile(dec_b1, 4).reshape(256, 1), 56, 56, 256, 8)
  d1 = _interleave(t1, 56, 64, 56)
  d1p = jnp.pad(d1[:, :111, :, :111], ((0, 0), (0, 1), (0, 0), (0, 1)))

  t2 = _convt(d1p, jnp.transpose(_pack_convt_w(dec_w2), (1, 0)),
              jnp.tile(dec_b2, 4).reshape(128, 1), 111, 111, 128, 3)
  d2s = jnp.pad(_interleave(t2, 111, 32, 111),
                ((0, 0), (1, 1), (0, 0), (1, 1)))

  w3f = jnp.transpose(dec_w3[0], (1, 2, 0)).reshape(1, 288)
  out = _conv3f(d2s, w3f, dec_b3.reshape(1, 1))
  x_recon = out.reshape(4, 1, 222, 222)
  return (x_recon, vq_loss)
